# baseline (speedup 1.0000x reference)
import jax
import jax.numpy as jnp
from jax.experimental import pallas as pl
from jax.experimental.pallas import tpu as pltpu

_GROUP_ORDER = 16
_FACTOR = 2
_SUB = _GROUP_ORDER // _FACTOR
_F = 96


def _dma_body(x_ref, o_ref, sem):
    copies = []
    for g in range(_SUB):
        copies.append(
            pltpu.make_async_copy(
                x_ref.at[:, pl.ds(_FACTOR * _F * g, _F)],
                o_ref.at[:, pl.ds(_F * g, _F)],
                sem,
            )
        )
    for cp in copies:
        cp.start()
    for cp in copies:
        cp.wait()


def kernel(x):
    B, C, H, W = x.shape
    return pl.pallas_call(
        _dma_body,
        in_specs=[pl.BlockSpec(memory_space=pltpu.MemorySpace.HBM)],
        out_specs=pl.BlockSpec(memory_space=pltpu.MemorySpace.HBM),
        scratch_shapes=[pltpu.SemaphoreType.DMA],
        out_shape=jax.ShapeDtypeStruct((B, _SUB * _F, H, W), jnp.float32),
    )(x)


# speedup vs baseline: 11.4889x; 11.4889x over previous
import jax
import jax.numpy as jnp
from jax.experimental import pallas as pl

_GROUP_ORDER = 16
_FACTOR = 2
_SUB = _GROUP_ORDER // _FACTOR
_F = 96


def _copy_body(in_ref, out_ref):
    out_ref[0] = in_ref[...]


def kernel(x):
    B, C, H, W = x.shape
    out = pl.pallas_call(
        _copy_body,
        grid=(B, _SUB),
        in_specs=[
            pl.BlockSpec((1, _F, H, W), lambda b, g: (b, _FACTOR * g, 0, 0))
        ],
        out_specs=pl.BlockSpec((1, 1, _F, H, W), lambda b, g: (b, g, 0, 0, 0)),
        out_shape=jax.ShapeDtypeStruct((B, _SUB, _F, H, W), jnp.float32),
    )(x)
    return out.reshape(B, _SUB * _F, H, W)


# speedup vs baseline: 14.7606x; 1.2848x over previous
import jax
import jax.numpy as jnp
from jax.experimental import pallas as pl

_GROUP_ORDER = 16
_FACTOR = 2
_SUB = _GROUP_ORDER // _FACTOR
_F = 96


def _copy_body(in_ref, out_ref):
    out_ref[...] = in_ref[0]


def kernel(x):
    B, C, H, W = x.shape
    xv = x.reshape(B, _GROUP_ORDER, _F, H, W)
    return pl.pallas_call(
        _copy_body,
        grid=(B, _SUB),
        in_specs=[
            pl.BlockSpec((1, 1, _F, H, W), lambda b, g: (b, _FACTOR * g, 0, 0, 0))
        ],
        out_specs=pl.BlockSpec((1, _F, H, W), lambda b, g: (b, g, 0, 0)),
        out_shape=jax.ShapeDtypeStruct((B, _SUB * _F, H, W), jnp.float32),
    )(xv)
